# baseline (speedup 1.0000x reference)
import jax
import jax.numpy as jnp
from jax import lax
from jax.experimental import pallas as pl
from jax.experimental.pallas import tpu as pltpu
from jax.experimental.pallas import tpu_sc as plsc

B, L, H = 1024, 200, 128
N = B * L
NW = 32
TPW = N // NW
RPW = B // NW
G = 64
NG = TPW // G
NBUF = 4
NH = H // 16


def _body(ids_hbm, wte_hbm, wpe_hbm, gamma_hbm, beta_hbm, out_hbm,
          ids_v, pos_v, wpe_sh,
          wte0, wte1, wte2, wte3, wpe0, wpe1, wpe2, wpe3,
          out0, out1, out2, out3, g_v, b_v,
          sw0, sw1, sw2, sw3, sp0, sp1, sp2, sp3, so0, so1, so2, so3):
    cid = lax.axis_index("c")
    sid = lax.axis_index("s")
    wid = sid * 2 + cid
    base = pl.multiple_of(wid * TPW, 8)

    wte_b, wpe_b, out_b = ([wte0, wte1, wte2, wte3], [wpe0, wpe1, wpe2, wpe3],
                           [out0, out1, out2, out3])
    sw, sp, so = [sw0, sw1, sw2, sw3], [sp0, sp1, sp2, sp3], [so0, so1, so2, so3]

    @pl.when(sid == 0)
    def _():
        pltpu.sync_copy(wpe_hbm, wpe_sh)

    pltpu.sync_copy(ids_hbm.at[pl.ds(base, TPW)], ids_v)
    pltpu.sync_copy(gamma_hbm, g_v)
    pltpu.sync_copy(beta_hbm, b_v)

    gs = [g_v[pl.ds(16 * i, 16)] for i in range(NH)]
    bs = [b_v[pl.ds(16 * i, 16)] for i in range(NH)]

    def pos_chunk(off, carry):
        ch = ids_v[pl.ds(off, 16)]
        m = (ch != 0).astype(jnp.int32)
        cs = jnp.cumsum(m)
        pos_v[pl.ds(off, 16)] = (carry + cs) * m
        return m

    def row_body(r, x):
        off0 = r * L
        carry = jnp.int32(0)
        for c in range(11):
            m = pos_chunk(pl.multiple_of(off0 + c * 16, 8), carry)
            carry = carry + jnp.sum(m)
        m = pos_chunk(pl.multiple_of(off0 + 176, 8), carry)
        first8 = jnp.where(lax.iota(jnp.int32, 16) < 8, m, 0)
        carry2 = carry + jnp.sum(first8)
        pos_chunk(pl.multiple_of(off0 + 184, 8), carry2)
        return x

    lax.fori_loop(0, RPW, row_body, jnp.int32(0))

    plsc.subcore_barrier()

    def fire(g, b):
        goff = pl.multiple_of(g * G, 8)
        pltpu.async_copy(wte_hbm.at[ids_v.at[pl.ds(goff, G)]], wte_b[b], sw[b])
        pltpu.async_copy(wpe_sh.at[pos_v.at[pl.ds(goff, G)]], wpe_b[b], sp[b])

    for b in range(NBUF):
        fire(jnp.int32(b), b)

    def compute(b, goff):
        wte_v, wpe_v, out_v = wte_b[b], wpe_b[b], out_b[b]

        def tok_body(t, y):
            e = [wte_v[t, pl.ds(16 * i, 16)] + wpe_v[t, pl.ds(16 * i, 16)]
                 for i in range(NH)]
            s1 = ((e[0] + e[1]) + (e[2] + e[3])) + ((e[4] + e[5]) + (e[6] + e[7]))
            sq = [v * v for v in e]
            s2 = ((sq[0] + sq[1]) + (sq[2] + sq[3])) + ((sq[4] + sq[5]) + (sq[6] + sq[7]))
            mean = jnp.sum(s1) * (1.0 / H)
            var = jnp.sum(s2) * (1.0 / H) - mean * mean
            xv = var + 1e-5
            xi = lax.bitcast_convert_type(xv, jnp.int32)
            yi = jnp.int32(0x5F3759DF) - lax.shift_right_arithmetic(xi, 1)
            r = lax.bitcast_convert_type(yi, jnp.float32)
            for _ in range(3):
                r = r * (1.5 - 0.5 * xv * r * r)
            for i in range(NH):
                out_v[t, pl.ds(16 * i, 16)] = (e[i] - mean) * r * gs[i] + bs[i]
            return y

        lax.fori_loop(0, G, tok_body, jnp.int32(0))

    def pair_body(p, x):
        for b in range(NBUF):
            g = p * NBUF + b
            goff = pl.multiple_of(g * G, 8)
            pltpu.make_async_copy(
                wte_hbm.at[ids_v.at[pl.ds(0, G)]], wte_b[b], sw[b]).wait()
            pltpu.make_async_copy(
                wpe_sh.at[pos_v.at[pl.ds(0, G)]], wpe_b[b], sp[b]).wait()

            @pl.when(g >= NBUF)
            def _():
                pltpu.make_async_copy(
                    out_b[b], out_hbm.at[pl.ds(base, G)], so[b]).wait()

            compute(b, goff)
            pltpu.async_copy(out_b[b], out_hbm.at[pl.ds(base + goff, G)], so[b])

            @pl.when(g + NBUF < NG)
            def _():
                fire(g + NBUF, b)
        return x

    lax.fori_loop(0, NG // NBUF, pair_body, jnp.int32(0))

    for b in range(NBUF):
        pltpu.make_async_copy(out_b[b], out_hbm.at[pl.ds(base, G)], so[b]).wait()


@jax.jit
def _run(ids, wte, wpe, gamma, beta):
    mesh = plsc.VectorSubcoreMesh(core_axis_name="c", subcore_axis_name="s")
    f = pl.kernel(
        _body,
        out_type=jax.ShapeDtypeStruct((N, H), jnp.float32),
        mesh=mesh,
        scratch_types=[
            pltpu.VMEM((TPW,), jnp.int32),
            pltpu.VMEM((TPW,), jnp.int32),
            pltpu.VMEM_SHARED((512, H), jnp.float32),
        ]
        + [pltpu.VMEM((G, H), jnp.float32)] * (3 * NBUF)
        + [
            pltpu.VMEM((H,), jnp.float32),
            pltpu.VMEM((H,), jnp.float32),
        ]
        + [pltpu.SemaphoreType.DMA] * (3 * NBUF),
        compiler_params=pltpu.CompilerParams(needs_layout_passes=False),
    )
    return f(ids, wte, wpe, gamma, beta)


def kernel(input_ids, wte, wpe, ln_gamma, ln_beta):
    ids = input_ids.reshape(-1).astype(jnp.int32)
    out = _run(ids, wte, wpe, ln_gamma, ln_beta)
    return out.reshape(input_ids.shape + (H,))


# speedup vs baseline: 1.0081x; 1.0081x over previous
import jax
import jax.numpy as jnp
from jax import lax
from jax.experimental import pallas as pl
from jax.experimental.pallas import tpu as pltpu
from jax.experimental.pallas import tpu_sc as plsc

B, L, H = 1024, 200, 128
N = B * L
NW = 32
TPW = N // NW
RPW = B // NW
G = 64
NG = TPW // G
NBUF = 4
NH = H // 16


def _body(ids_hbm, wte_hbm, wpe_hbm, gamma_hbm, beta_hbm, out_hbm,
          ids_v, pos_v, wpe_sh,
          wte0, wte1, wte2, wte3, wpe0, wpe1, wpe2, wpe3,
          out0, out1, out2, out3, g_v, b_v,
          sw0, sw1, sw2, sw3, sp0, sp1, sp2, sp3, so0, so1, so2, so3):
    cid = lax.axis_index("c")
    sid = lax.axis_index("s")
    wid = sid * 2 + cid
    base = pl.multiple_of(wid * TPW, 8)

    wte_b, wpe_b, out_b = ([wte0, wte1, wte2, wte3], [wpe0, wpe1, wpe2, wpe3],
                           [out0, out1, out2, out3])
    sw, sp, so = [sw0, sw1, sw2, sw3], [sp0, sp1, sp2, sp3], [so0, so1, so2, so3]

    @pl.when(sid == 0)
    def _():
        pltpu.sync_copy(wpe_hbm, wpe_sh)

    pltpu.sync_copy(ids_hbm.at[pl.ds(base, TPW)], ids_v)
    pltpu.sync_copy(gamma_hbm, g_v)
    pltpu.sync_copy(beta_hbm, b_v)

    gs = [g_v[pl.ds(16 * i, 16)] for i in range(NH)]
    bs = [b_v[pl.ds(16 * i, 16)] for i in range(NH)]

    def pos_chunk(off, carry):
        ch = ids_v[pl.ds(off, 16)]
        m = (ch != 0).astype(jnp.int32)
        cs = jnp.cumsum(m)
        pos_v[pl.ds(off, 16)] = (carry + cs) * m
        return m

    def row_body(r, x):
        off0 = r * L
        carry = jnp.int32(0)
        for c in range(11):
            m = pos_chunk(pl.multiple_of(off0 + c * 16, 8), carry)
            carry = carry + jnp.sum(m)
        m = pos_chunk(pl.multiple_of(off0 + 176, 8), carry)
        first8 = jnp.where(lax.iota(jnp.int32, 16) < 8, m, 0)
        carry2 = carry + jnp.sum(first8)
        pos_chunk(pl.multiple_of(off0 + 184, 8), carry2)
        return x

    def fire_wte(g, b):
        goff = pl.multiple_of(g * G, 8)
        pltpu.async_copy(wte_hbm.at[ids_v.at[pl.ds(goff, G)]], wte_b[b], sw[b])

    def fire_wpe(g, b):
        goff = pl.multiple_of(g * G, 8)
        pltpu.async_copy(wpe_sh.at[pos_v.at[pl.ds(goff, G)]], wpe_b[b], sp[b])

    def fire(g, b):
        fire_wte(g, b)
        fire_wpe(g, b)

    for b in range(NBUF):
        fire_wte(jnp.int32(b), b)

    lax.fori_loop(0, RPW, row_body, jnp.int32(0))

    plsc.subcore_barrier()

    for b in range(NBUF):
        fire_wpe(jnp.int32(b), b)

    def compute(b, goff):
        wte_v, wpe_v, out_v = wte_b[b], wpe_b[b], out_b[b]

        def tok_body(t, y):
            e = [wte_v[t, pl.ds(16 * i, 16)] + wpe_v[t, pl.ds(16 * i, 16)]
                 for i in range(NH)]
            s1 = ((e[0] + e[1]) + (e[2] + e[3])) + ((e[4] + e[5]) + (e[6] + e[7]))
            sq = [v * v for v in e]
            s2 = ((sq[0] + sq[1]) + (sq[2] + sq[3])) + ((sq[4] + sq[5]) + (sq[6] + sq[7]))
            mean = jnp.sum(s1) * (1.0 / H)
            var = jnp.sum(s2) * (1.0 / H) - mean * mean
            xv = var + 1e-5
            xi = lax.bitcast_convert_type(xv, jnp.int32)
            yi = jnp.int32(0x5F3759DF) - lax.shift_right_arithmetic(xi, 1)
            r = lax.bitcast_convert_type(yi, jnp.float32)
            for _ in range(3):
                r = r * (1.5 - 0.5 * xv * r * r)
            for i in range(NH):
                out_v[t, pl.ds(16 * i, 16)] = (e[i] - mean) * r * gs[i] + bs[i]
            return y

        lax.fori_loop(0, G, tok_body, jnp.int32(0))

    def pair_body(p, x):
        for b in range(NBUF):
            g = p * NBUF + b
            goff = pl.multiple_of(g * G, 8)
            pltpu.make_async_copy(
                wte_hbm.at[ids_v.at[pl.ds(0, G)]], wte_b[b], sw[b]).wait()
            pltpu.make_async_copy(
                wpe_sh.at[pos_v.at[pl.ds(0, G)]], wpe_b[b], sp[b]).wait()

            @pl.when(g >= NBUF)
            def _():
                pltpu.make_async_copy(
                    out_b[b], out_hbm.at[pl.ds(base, G)], so[b]).wait()

            compute(b, goff)
            pltpu.async_copy(out_b[b], out_hbm.at[pl.ds(base + goff, G)], so[b])

            @pl.when(g + NBUF < NG)
            def _():
                fire(g + NBUF, b)
        return x

    lax.fori_loop(0, NG // NBUF, pair_body, jnp.int32(0))

    for b in range(NBUF):
        pltpu.make_async_copy(out_b[b], out_hbm.at[pl.ds(base, G)], so[b]).wait()


@jax.jit
def _run(ids, wte, wpe, gamma, beta):
    mesh = plsc.VectorSubcoreMesh(core_axis_name="c", subcore_axis_name="s")
    f = pl.kernel(
        _body,
        out_type=jax.ShapeDtypeStruct((N, H), jnp.float32),
        mesh=mesh,
        scratch_types=[
            pltpu.VMEM((TPW,), jnp.int32),
            pltpu.VMEM((TPW,), jnp.int32),
            pltpu.VMEM_SHARED((512, H), jnp.float32),
        ]
        + [pltpu.VMEM((G, H), jnp.float32)] * (3 * NBUF)
        + [
            pltpu.VMEM((H,), jnp.float32),
            pltpu.VMEM((H,), jnp.float32),
        ]
        + [pltpu.SemaphoreType.DMA] * (3 * NBUF),
        compiler_params=pltpu.CompilerParams(needs_layout_passes=False),
    )
    return f(ids, wte, wpe, gamma, beta)


def kernel(input_ids, wte, wpe, ln_gamma, ln_beta):
    ids = input_ids.reshape(-1).astype(jnp.int32)
    out = _run(ids, wte, wpe, ln_gamma, ln_beta)
    return out.reshape(input_ids.shape + (H,))
